# baseline (speedup 1.0000x reference)
import functools

import jax
import jax.numpy as jnp
from jax import lax
from jax.experimental import pallas as pl
from jax.experimental.pallas import tpu as pltpu
from jax.experimental.pallas import tpu_sc as plsc

NC = 2
NS = 16
L = 16


def _sc_sigma(m, idx, kvec, x):
    E, D = m.shape
    N = x.shape[0]
    HALF = N // NC
    T = 128
    NBUF = 5
    AROWS = HALF + 8
    RPW_A = ((HALF // NS) + 7) // 8 * 8
    RPW_B = HALF - (NS - 1) * RPW_A

    mesh = plsc.VectorSubcoreMesh(core_axis_name="c", subcore_axis_name="s")

    @functools.partial(
        pl.kernel,
        out_type=jax.ShapeDtypeStruct((N, D), jnp.float32),
        mesh=mesh,
        scratch_types=[
            [pltpu.VMEM((T,), jnp.int32) for _ in range(NBUF)],
            [pltpu.VMEM((T, D), jnp.float32) for _ in range(NBUF)],
            [pltpu.SemaphoreType.DMA for _ in range(NBUF)],
            [pltpu.SemaphoreType.DMA for _ in range(NBUF)],
            pltpu.VMEM((L,), jnp.int32),
            pltpu.VMEM_SHARED((AROWS, D), jnp.float32),
        ],
    )
    def body(m_hbm, idx_hbm, k_hbm, x_hbm, out_hbm,
             idx_v, m_v, isem, msem, k_v, acc_sh):
        c = lax.axis_index("c")
        s = lax.axis_index("s")

        pltpu.sync_copy(k_hbm, k_v)
        k = k_v[...][0]

        lo_sc = jnp.where(c == 0, 0, (k // 8) * 8)
        hi_sc = jnp.where(c == 0, jnp.minimum((k + 7) // 8 * 8, E), E)
        cnt = hi_sc - lo_sc
        per8 = ((cnt + NS - 1) // NS + 7) // 8 * 8
        sub_lo = lo_sc + s * per8
        sub_hi = jnp.minimum(sub_lo + per8, hi_sc)
        nt = jnp.maximum((sub_hi - sub_lo + T - 1) // T, 0)
        base = c * HALF

        def load_tile(t, b):
            e_c = pl.multiple_of(jnp.minimum(sub_lo + t * T, E - T), 8)
            pltpu.async_copy(idx_hbm.at[pl.ds(e_c, T)], idx_v[b], isem[b])
            pltpu.async_copy(m_hbm.at[pl.ds(e_c, T)], m_v[b], msem[b])

        for b in range(NBUF):
            @pl.when(b < nt)
            def _():
                load_tile(b, b)

        off = pl.multiple_of(s * RPW_A, 8)
        xoff = pl.multiple_of(base + s * RPW_A, 8)

        @pl.when(s < NS - 1)
        def _():
            pltpu.sync_copy(x_hbm.at[pl.ds(xoff, RPW_A)],
                            acc_sh.at[pl.ds(off, RPW_A)])

        @pl.when(s == NS - 1)
        def _():
            pltpu.sync_copy(
                x_hbm.at[pl.ds(pl.multiple_of(base + (NS - 1) * RPW_A, 8),
                               RPW_B)],
                acc_sh.at[pl.ds((NS - 1) * RPW_A, RPW_B)])

        plsc.subcore_barrier()

        def process_tile(t, b):
            e_c = pl.multiple_of(jnp.minimum(sub_lo + t * T, E - T), 8)
            e_l = sub_lo + t * T
            pltpu.make_async_copy(
                idx_hbm.at[pl.ds(e_c, T)], idx_v[b], isem[b]).wait()
            pltpu.make_async_copy(
                m_hbm.at[pl.ds(e_c, T)], m_v[b], msem[b]).wait()
            hi_keep = jnp.minimum(e_l + T, sub_hi)
            for j in range(T // L):
                pos = e_c + j * L + lax.iota(jnp.int32, L)
                v = idx_v[b][pl.ds(j * L, L)]
                local = v - base
                keep = ((local >= 0) & (local < HALF)
                        & (pos >= e_l) & (pos < hi_keep))
                idx_v[b][pl.ds(j * L, L)] = jnp.where(keep, local, HALF)
            pltpu.sync_copy(m_v[b], acc_sh.at[idx_v[b]], add=True)

            @pl.when(t + NBUF < nt)
            def _():
                load_tile(t + NBUF, b)

        def group_body(g, carry):
            for b in range(NBUF):
                @pl.when(g * NBUF + b < nt)
                def _():
                    process_tile(g * NBUF + b, b)
            return carry

        lax.fori_loop(0, (nt + NBUF - 1) // NBUF, group_body, 0)
        plsc.subcore_barrier()

        @pl.when(s < NS - 1)
        def _():
            pltpu.sync_copy(acc_sh.at[pl.ds(off, RPW_A)],
                            out_hbm.at[pl.ds(xoff, RPW_A)])

        @pl.when(s == NS - 1)
        def _():
            pltpu.sync_copy(
                acc_sh.at[pl.ds((NS - 1) * RPW_A, RPW_B)],
                out_hbm.at[pl.ds(pl.multiple_of(base + (NS - 1) * RPW_A, 8),
                                 RPW_B)])

    return body(m, idx, kvec, x)


def kernel(m, i, n, x):
    N = x.shape[0]
    idx = jnp.asarray(i, jnp.int32)
    k = jnp.sum((idx < N // NC).astype(jnp.int32)).astype(jnp.int32)
    kvec = jnp.broadcast_to(k, (L,))
    return _sc_sigma(m, idx, kvec, x)


# speedup vs baseline: 1.0092x; 1.0092x over previous
import functools

import jax
import jax.numpy as jnp
from jax import lax
from jax.experimental import pallas as pl
from jax.experimental.pallas import tpu as pltpu
from jax.experimental.pallas import tpu_sc as plsc

NC = 2
NS = 16
L = 16


def _sc_sigma(m, idx, kvec, x):
    E, D = m.shape
    N = x.shape[0]
    HALF = N // NC
    T = 48
    NBUF = 8
    AROWS = HALF + 8
    RPW_A = ((HALF // NS) + 7) // 8 * 8
    RPW_B = HALF - (NS - 1) * RPW_A

    mesh = plsc.VectorSubcoreMesh(core_axis_name="c", subcore_axis_name="s")

    @functools.partial(
        pl.kernel,
        out_type=jax.ShapeDtypeStruct((N, D), jnp.float32),
        mesh=mesh,
        scratch_types=[
            [pltpu.VMEM((T,), jnp.int32) for _ in range(NBUF)],
            [pltpu.VMEM((T, D), jnp.float32) for _ in range(NBUF)],
            [pltpu.SemaphoreType.DMA for _ in range(NBUF)],
            [pltpu.SemaphoreType.DMA for _ in range(NBUF)],
            pltpu.VMEM((L,), jnp.int32),
            pltpu.VMEM_SHARED((AROWS, D), jnp.float32),
        ],
    )
    def body(m_hbm, idx_hbm, k_hbm, x_hbm, out_hbm,
             idx_v, m_v, isem, msem, k_v, acc_sh):
        c = lax.axis_index("c")
        s = lax.axis_index("s")

        pltpu.sync_copy(k_hbm, k_v)
        k = k_v[...][0]

        lo_sc = jnp.where(c == 0, 0, (k // 8) * 8)
        hi_sc = jnp.where(c == 0, jnp.minimum((k + 7) // 8 * 8, E), E)
        cnt = hi_sc - lo_sc
        per8 = ((cnt + NS - 1) // NS + 7) // 8 * 8
        sub_lo = lo_sc + s * per8
        sub_hi = jnp.minimum(sub_lo + per8, hi_sc)
        nt = jnp.maximum((sub_hi - sub_lo + T - 1) // T, 0)
        base = c * HALF

        def load_tile(t, b):
            e_c = pl.multiple_of(jnp.minimum(sub_lo + t * T, E - T), 8)
            pltpu.async_copy(idx_hbm.at[pl.ds(e_c, T)], idx_v[b], isem[b])
            pltpu.async_copy(m_hbm.at[pl.ds(e_c, T)], m_v[b], msem[b])

        for b in range(NBUF):
            @pl.when(b < nt)
            def _():
                load_tile(b, b)

        off = pl.multiple_of(s * RPW_A, 8)
        xoff = pl.multiple_of(base + s * RPW_A, 8)

        @pl.when(s < NS - 1)
        def _():
            pltpu.sync_copy(x_hbm.at[pl.ds(xoff, RPW_A)],
                            acc_sh.at[pl.ds(off, RPW_A)])

        @pl.when(s == NS - 1)
        def _():
            pltpu.sync_copy(
                x_hbm.at[pl.ds(pl.multiple_of(base + (NS - 1) * RPW_A, 8),
                               RPW_B)],
                acc_sh.at[pl.ds((NS - 1) * RPW_A, RPW_B)])

        plsc.subcore_barrier()

        def process_tile(t, b):
            e_c = pl.multiple_of(jnp.minimum(sub_lo + t * T, E - T), 8)
            e_l = sub_lo + t * T
            pltpu.make_async_copy(
                idx_hbm.at[pl.ds(e_c, T)], idx_v[b], isem[b]).wait()
            pltpu.make_async_copy(
                m_hbm.at[pl.ds(e_c, T)], m_v[b], msem[b]).wait()
            hi_keep = jnp.minimum(e_l + T, sub_hi)
            for j in range(T // L):
                pos = e_c + j * L + lax.iota(jnp.int32, L)
                v = idx_v[b][pl.ds(j * L, L)]
                local = v - base
                keep = ((local >= 0) & (local < HALF)
                        & (pos >= e_l) & (pos < hi_keep))
                idx_v[b][pl.ds(j * L, L)] = jnp.where(keep, local, HALF)
            pltpu.sync_copy(m_v[b], acc_sh.at[idx_v[b]], add=True)

            @pl.when(t + NBUF < nt)
            def _():
                load_tile(t + NBUF, b)

        def group_body(g, carry):
            for b in range(NBUF):
                @pl.when(g * NBUF + b < nt)
                def _():
                    process_tile(g * NBUF + b, b)
            return carry

        lax.fori_loop(0, (nt + NBUF - 1) // NBUF, group_body, 0)
        plsc.subcore_barrier()

        @pl.when(s < NS - 1)
        def _():
            pltpu.sync_copy(acc_sh.at[pl.ds(off, RPW_A)],
                            out_hbm.at[pl.ds(xoff, RPW_A)])

        @pl.when(s == NS - 1)
        def _():
            pltpu.sync_copy(
                acc_sh.at[pl.ds((NS - 1) * RPW_A, RPW_B)],
                out_hbm.at[pl.ds(pl.multiple_of(base + (NS - 1) * RPW_A, 8),
                                 RPW_B)])

    return body(m, idx, kvec, x)


def kernel(m, i, n, x):
    N = x.shape[0]
    idx = jnp.asarray(i, jnp.int32)
    k = jnp.sum((idx < N // NC).astype(jnp.int32)).astype(jnp.int32)
    kvec = jnp.broadcast_to(k, (L,))
    return _sc_sigma(m, idx, kvec, x)
